# baseline (speedup 1.0000x reference)
import jax
import jax.numpy as jnp
from jax import lax
from jax.experimental import pallas as pl
from jax.experimental.pallas import tpu as pltpu
from jax.experimental.pallas import tpu_sc as plsc

_NC = 2
_NS = 16
_NW = _NC * _NS
_B = 80
_GB = 32
_CPW = 128
_RING = 4

_SC_PARAMS = pltpu.CompilerParams(use_tc_tiling_on_sc=False)


def _make_agg(N, D):
    npw = N // _NS
    Np = N + 8
    ngroup = _CPW // _GB
    mesh = plsc.VectorSubcoreMesh(core_axis_name="c", subcore_axis_name="s")
    out_type = [jax.ShapeDtypeStruct((N, D), jnp.float32),
                jax.ShapeDtypeStruct((N, D), jnp.float32)]
    scratch = (
        [pltpu.VMEM((_GB, _B), jnp.int32)] * 2
        + [pltpu.VMEM((_B, D), jnp.float32)] * _RING
        + [pltpu.VMEM_SHARED((Np, D), jnp.float32)]
        + [pltpu.SemaphoreType.DMA] * (2 * _RING)
    )

    def body(src_hbm, dst_hbm, h_hbm, z_nd, out0, out1,
             src_v, dst_v, *rest):
        bufs = rest[:_RING]
        acc = rest[_RING]
        gsems = rest[_RING + 1:2 * _RING + 1]
        ssems = rest[2 * _RING + 1:]
        cid = lax.axis_index("c")
        sid = lax.axis_index("s")
        wid = cid * _NS + sid

        pltpu.sync_copy(z_nd.at[pl.ds(sid * npw, npw)],
                        acc.at[pl.ds(sid * npw, npw)])
        plsc.subcore_barrier()

        def gather(j, k):
            pltpu.async_copy(h_hbm.at[src_v.at[j]], bufs[k], gsems[k])

        def wait_g(k):
            pltpu.make_async_copy(h_hbm.at[src_v.at[0]], bufs[k],
                                  gsems[k]).wait()

        def scat(j, k):
            pltpu.async_copy(bufs[k], acc.at[dst_v.at[j]], ssems[k], add=True)

        def wait_s(k):
            pltpu.make_async_copy(bufs[k], acc.at[dst_v.at[0]],
                                  ssems[k]).wait()

        @pl.loop(0, ngroup)
        def _(g):
            base = wid * _CPW + g * _GB
            pltpu.sync_copy(src_hbm.at[pl.ds(base, _GB)], src_v)
            pltpu.sync_copy(dst_hbm.at[pl.ds(base, _GB)], dst_v)
            for k in range(_RING):
                gather(k, k)

            @pl.loop(0, _GB // _RING - 1)
            def _(t):
                j = _RING * t
                for k in range(_RING):
                    wait_g(k)
                    scat(j + k, k)
                for k in range(_RING):
                    wait_s(k)
                    gather(j + _RING + k, k)

            for k in range(_RING):
                wait_g(k)
                scat(_GB - _RING + k, k)
            for k in range(_RING):
                wait_s(k)

        plsc.subcore_barrier()

        @pl.when(cid == 0)
        def _():
            pltpu.sync_copy(acc.at[pl.ds(sid * npw, npw)],
                            out0.at[pl.ds(sid * npw, npw)])

        @pl.when(cid == 1)
        def _():
            pltpu.sync_copy(acc.at[pl.ds(sid * npw, npw)],
                            out1.at[pl.ds(sid * npw, npw)])

    return pl.kernel(body, out_type=out_type, mesh=mesh,
                     scratch_types=scratch, compiler_params=_SC_PARAMS)


def _make_cnt(N):
    npw = N // _NS
    Np = N + 8
    ngroup = _CPW // _GB
    mesh = plsc.VectorSubcoreMesh(core_axis_name="c", subcore_axis_name="s")
    out_type = [jax.ShapeDtypeStruct((N, 16), jnp.float32),
                jax.ShapeDtypeStruct((N, 16), jnp.float32)]
    scratch = (
        [pltpu.VMEM((_GB, _B), jnp.int32),
         pltpu.VMEM((_B, 16), jnp.float32),
         pltpu.VMEM_SHARED((Np, 16), jnp.float32)]
        + [pltpu.SemaphoreType.DMA] * _RING
    )

    def body(dst_hbm, z16, ones_hbm, out0, out1, dst_v, ones_v, cnt, *sems):
        cid = lax.axis_index("c")
        sid = lax.axis_index("s")
        wid = cid * _NS + sid

        pltpu.sync_copy(z16.at[pl.ds(sid * npw, npw)],
                        cnt.at[pl.ds(sid * npw, npw)])
        pltpu.sync_copy(ones_hbm, ones_v)
        plsc.subcore_barrier()

        def scat(j, k):
            pltpu.async_copy(ones_v, cnt.at[dst_v.at[j]], sems[k], add=True)

        def wait_s(k):
            pltpu.make_async_copy(ones_v, cnt.at[dst_v.at[0]],
                                  sems[k]).wait()

        @pl.loop(0, ngroup)
        def _(g):
            base = wid * _CPW + g * _GB
            pltpu.sync_copy(dst_hbm.at[pl.ds(base, _GB)], dst_v)
            for k in range(_RING):
                scat(k, k)

            @pl.loop(0, _GB // _RING - 1)
            def _(t):
                j = _RING * t
                for k in range(_RING):
                    wait_s(k)
                    scat(j + _RING + k, k)

            for k in range(_RING):
                wait_s(k)

        plsc.subcore_barrier()

        @pl.when(cid == 0)
        def _():
            pltpu.sync_copy(cnt.at[pl.ds(sid * npw, npw)],
                            out0.at[pl.ds(sid * npw, npw)])

        @pl.when(cid == 1)
        def _():
            pltpu.sync_copy(cnt.at[pl.ds(sid * npw, npw)],
                            out1.at[pl.ds(sid * npw, npw)])

    return pl.kernel(body, out_type=out_type, mesh=mesh,
                     scratch_types=scratch, compiler_params=_SC_PARAMS)


def _mm_body(a0_ref, a1_ref, c0_ref, c1_ref, h_ref, wl_ref, wr_ref, bl_ref,
             pre_ref, act_ref):
    s = a0_ref[...] + a1_ref[...]
    cnt = c0_ref[...] + c1_ref[...]
    c = jnp.maximum(cnt[:, 0:1], 1.0)
    m = s / c
    pre = (jnp.dot(m, wl_ref[...], preferred_element_type=jnp.float32)
           + jnp.dot(h_ref[...], wr_ref[...], preferred_element_type=jnp.float32)
           + bl_ref[...])
    pre_ref[...] = pre
    act_ref[...] = jnp.maximum(pre, 0.0)


def _mm(a0, a1, c0, c1, h, Wl, Wr, bl):
    N, D = h.shape
    R = 1000
    return pl.pallas_call(
        _mm_body,
        grid=(N // R,),
        in_specs=[
            pl.BlockSpec((R, D), lambda i: (i, 0)),
            pl.BlockSpec((R, D), lambda i: (i, 0)),
            pl.BlockSpec((R, 16), lambda i: (i, 0)),
            pl.BlockSpec((R, 16), lambda i: (i, 0)),
            pl.BlockSpec((R, D), lambda i: (i, 0)),
            pl.BlockSpec((D, D), lambda i: (0, 0)),
            pl.BlockSpec((D, D), lambda i: (0, 0)),
            pl.BlockSpec((1, D), lambda i: (0, 0)),
        ],
        out_specs=[pl.BlockSpec((R, D), lambda i: (i, 0)),
                   pl.BlockSpec((R, D), lambda i: (i, 0))],
        out_shape=[jax.ShapeDtypeStruct((N, D), jnp.float32),
                   jax.ShapeDtypeStruct((N, D), jnp.float32)],
    )(a0, a1, c0, c1, h, Wl, Wr, bl.reshape(1, D))


def kernel(x, edge_index, Wl0, bl0, Wr0, Wl1, bl1, Wr1, Wl2, bl2, Wr2):
    N, D = x.shape
    E = edge_index.shape[1]
    E_pad = _CPW * _NW * _B
    pad = E_pad - E
    src = jnp.concatenate(
        [edge_index[0], jnp.zeros((pad,), jnp.int32)]).reshape(E_pad // _B, _B)
    dst = jnp.concatenate(
        [edge_index[1], jnp.full((pad,), N, jnp.int32)]).reshape(E_pad // _B, _B)
    z_nd = jnp.zeros((N, D), jnp.float32)
    z16 = jnp.zeros((N, 16), jnp.float32)
    ones_b = jnp.ones((_B, 16), jnp.float32)

    agg = _make_agg(N, D)
    cnt = _make_cnt(N)

    c0, c1 = cnt(dst, z16, ones_b)
    a0, a1 = agg(src, dst, x, z_nd)
    pre0, h1 = _mm(a0, a1, c0, c1, x, Wl0, Wr0, bl0)
    a0, a1 = agg(src, dst, h1, z_nd)
    pre1, h2 = _mm(a0, a1, c0, c1, h1, Wl1, Wr1, bl1)
    a0, a1 = agg(src, dst, h2, z_nd)
    pre2, _ = _mm(a0, a1, c0, c1, h2, Wl2, Wr2, bl2)
    return (pre2, pre1)


# speedup vs baseline: 3.4230x; 3.4230x over previous
import jax
import jax.numpy as jnp
from jax import lax
from jax.experimental import pallas as pl
from jax.experimental.pallas import tpu as pltpu
from jax.experimental.pallas import tpu_sc as plsc

_NC = 2
_NS = 16
_NW = _NC * _NS
_B = 128
_GB = 20
_NG = 4

_SC_PARAMS = pltpu.CompilerParams(use_tc_tiling_on_sc=False)


def _make_agg(N, D, n_real):
    npw = N // _NS
    cpw = _NG * _GB
    mesh = plsc.VectorSubcoreMesh(core_axis_name="c", subcore_axis_name="s")
    out_type = [jax.ShapeDtypeStruct((N, D), jnp.float32),
                jax.ShapeDtypeStruct((N, D), jnp.float32)]
    scratch = (
        [pltpu.VMEM((_GB, _B), jnp.int32)] * 2
        + [pltpu.VMEM((_B, D), jnp.float32)] * 2
        + [pltpu.VMEM_SHARED((N, D), jnp.float32)]
        + [pltpu.SemaphoreType.DMA] * 2
    )

    def body(src_hbm, dst_hbm, h_hbm, z_nd, out0, out1,
             src_v, dst_v, buf0, buf1, acc, sem0, sem1):
        cid = lax.axis_index("c")
        sid = lax.axis_index("s")
        wid = cid * _NS + sid

        pltpu.sync_copy(z_nd.at[pl.ds(sid * npw, npw)],
                        acc.at[pl.ds(sid * npw, npw)])
        plsc.subcore_barrier()

        def gather(j, buf, sem):
            pltpu.async_copy(h_hbm.at[src_v.at[j]], buf, sem)

        def wait_g(buf, sem):
            pltpu.make_async_copy(h_hbm.at[src_v.at[0]], buf, sem).wait()

        def scat(j, buf):
            pltpu.sync_copy(buf, acc.at[dst_v.at[j]], add=True)

        @pl.loop(0, _NG)
        def _(g):
            base = wid * cpw + g * _GB

            @pl.when(base < n_real)
            def _():
                pltpu.sync_copy(src_hbm.at[pl.ds(base, _GB)], src_v)
                pltpu.sync_copy(dst_hbm.at[pl.ds(base, _GB)], dst_v)
                gather(0, buf0, sem0)
                gather(1, buf1, sem1)

                @pl.loop(0, (_GB - 2) // 2)
                def _(t):
                    j = 2 * t
                    wait_g(buf0, sem0)
                    scat(j, buf0)
                    gather(j + 2, buf0, sem0)
                    wait_g(buf1, sem1)
                    scat(j + 1, buf1)
                    gather(j + 3, buf1, sem1)

                wait_g(buf0, sem0)
                scat(_GB - 2, buf0)
                wait_g(buf1, sem1)
                scat(_GB - 1, buf1)

        plsc.subcore_barrier()

        @pl.when(cid == 0)
        def _():
            pltpu.sync_copy(acc.at[pl.ds(sid * npw, npw)],
                            out0.at[pl.ds(sid * npw, npw)])

        @pl.when(cid == 1)
        def _():
            pltpu.sync_copy(acc.at[pl.ds(sid * npw, npw)],
                            out1.at[pl.ds(sid * npw, npw)])

    return pl.kernel(body, out_type=out_type, mesh=mesh,
                     scratch_types=scratch, compiler_params=_SC_PARAMS)


def _make_cnt(N, n_real):
    npw = N // _NS
    cpw = _NG * _GB
    mesh = plsc.VectorSubcoreMesh(core_axis_name="c", subcore_axis_name="s")
    out_type = [jax.ShapeDtypeStruct((N, 16), jnp.float32),
                jax.ShapeDtypeStruct((N, 16), jnp.float32)]
    scratch = (
        [pltpu.VMEM((_GB, _B), jnp.int32),
         pltpu.VMEM((_B, 16), jnp.float32),
         pltpu.VMEM_SHARED((N, 16), jnp.float32)]
        + [pltpu.SemaphoreType.DMA] * 2
    )

    def body(dst_hbm, z16, ones_hbm, out0, out1, dst_v, ones_v, cnt,
             sem0, sem1):
        cid = lax.axis_index("c")
        sid = lax.axis_index("s")
        wid = cid * _NS + sid

        pltpu.sync_copy(z16.at[pl.ds(sid * npw, npw)],
                        cnt.at[pl.ds(sid * npw, npw)])
        pltpu.sync_copy(ones_hbm, ones_v)
        plsc.subcore_barrier()

        def scat(j, sem):
            pltpu.async_copy(ones_v, cnt.at[dst_v.at[j]], sem, add=True)

        def wait_s(sem):
            pltpu.make_async_copy(ones_v, cnt.at[dst_v.at[0]], sem).wait()

        @pl.loop(0, _NG)
        def _(g):
            base = wid * cpw + g * _GB

            @pl.when(base < n_real)
            def _():
                pltpu.sync_copy(dst_hbm.at[pl.ds(base, _GB)], dst_v)
                scat(0, sem0)
                scat(1, sem1)

                @pl.loop(0, (_GB - 2) // 2)
                def _(t):
                    j = 2 * t
                    wait_s(sem0)
                    scat(j + 2, sem0)
                    wait_s(sem1)
                    scat(j + 3, sem1)

                wait_s(sem0)
                wait_s(sem1)

        plsc.subcore_barrier()

        @pl.when(cid == 0)
        def _():
            pltpu.sync_copy(cnt.at[pl.ds(sid * npw, npw)],
                            out0.at[pl.ds(sid * npw, npw)])

        @pl.when(cid == 1)
        def _():
            pltpu.sync_copy(cnt.at[pl.ds(sid * npw, npw)],
                            out1.at[pl.ds(sid * npw, npw)])

    return pl.kernel(body, out_type=out_type, mesh=mesh,
                     scratch_types=scratch, compiler_params=_SC_PARAMS)


def _mm_body(a0_ref, a1_ref, c0_ref, c1_ref, h_ref, wl_ref, wr_ref, bl_ref,
             pre_ref, act_ref):
    s = a0_ref[...] + a1_ref[...]
    cnt = c0_ref[...] + c1_ref[...]
    c = jnp.maximum(cnt[:, 0:1], 1.0)
    m = s / c
    pre = (jnp.dot(m, wl_ref[...], preferred_element_type=jnp.float32)
           + jnp.dot(h_ref[...], wr_ref[...], preferred_element_type=jnp.float32)
           + bl_ref[...])
    pre_ref[...] = pre
    act_ref[...] = jnp.maximum(pre, 0.0)


def _mm(a0, a1, c0, c1, h, Wl, Wr, bl):
    N, D = h.shape
    R = 1000
    return pl.pallas_call(
        _mm_body,
        grid=(N // R,),
        in_specs=[
            pl.BlockSpec((R, D), lambda i: (i, 0)),
            pl.BlockSpec((R, D), lambda i: (i, 0)),
            pl.BlockSpec((R, 16), lambda i: (i, 0)),
            pl.BlockSpec((R, 16), lambda i: (i, 0)),
            pl.BlockSpec((R, D), lambda i: (i, 0)),
            pl.BlockSpec((D, D), lambda i: (0, 0)),
            pl.BlockSpec((D, D), lambda i: (0, 0)),
            pl.BlockSpec((1, D), lambda i: (0, 0)),
        ],
        out_specs=[pl.BlockSpec((R, D), lambda i: (i, 0)),
                   pl.BlockSpec((R, D), lambda i: (i, 0))],
        out_shape=[jax.ShapeDtypeStruct((N, D), jnp.float32),
                   jax.ShapeDtypeStruct((N, D), jnp.float32)],
    )(a0, a1, c0, c1, h, Wl, Wr, bl.reshape(1, D))


def kernel(x, edge_index, Wl0, bl0, Wr0, Wl1, bl1, Wr1, Wl2, bl2, Wr2):
    N, D = x.shape
    E = edge_index.shape[1]
    n_real = E // _B
    rows_pad = _NW * _NG * _GB
    pad = rows_pad * _B - E
    src = jnp.concatenate(
        [edge_index[0], jnp.zeros((pad,), jnp.int32)]).reshape(rows_pad, _B)
    dst = jnp.concatenate(
        [edge_index[1], jnp.zeros((pad,), jnp.int32)]).reshape(rows_pad, _B)
    z_nd = jnp.zeros((N, D), jnp.float32)
    z16 = jnp.zeros((N, 16), jnp.float32)
    ones_b = jnp.ones((_B, 16), jnp.float32)

    agg = _make_agg(N, D, n_real)
    cnt = _make_cnt(N, n_real)

    c0, c1 = cnt(dst, z16, ones_b)
    a0, a1 = agg(src, dst, x, z_nd)
    pre0, h1 = _mm(a0, a1, c0, c1, x, Wl0, Wr0, bl0)
    a0, a1 = agg(src, dst, h1, z_nd)
    pre1, h2 = _mm(a0, a1, c0, c1, h1, Wl1, Wr1, bl1)
    a0, a1 = agg(src, dst, h2, z_nd)
    pre2, _ = _mm(a0, a1, c0, c1, h2, Wl2, Wr2, bl2)
    return (pre2, pre1)


# speedup vs baseline: 3.5442x; 1.0354x over previous
import jax
import jax.numpy as jnp
from jax import lax
from jax.experimental import pallas as pl
from jax.experimental.pallas import tpu as pltpu
from jax.experimental.pallas import tpu_sc as plsc

_NC = 2
_NS = 16
_NW = _NC * _NS
_B = 128
_GB = 20
_NG = 4

_SC_PARAMS = pltpu.CompilerParams(use_tc_tiling_on_sc=False)


def _zero_vmem(buf, rows, width):
    z = jnp.zeros((16,), jnp.float32)

    @pl.loop(0, rows)
    def _(r):
        for k in range(width // 16):
            buf[r, pl.ds(16 * k, 16)] = z


def _copy_tiles(src_buf, dst_ref, base, total, step):
    nfull = total // step
    rem = total - nfull * step

    @pl.loop(0, nfull)
    def _(i):
        pltpu.sync_copy(src_buf, dst_ref.at[pl.ds(base + i * step, step)])

    if rem:
        pltpu.sync_copy(src_buf.at[pl.ds(0, rem)],
                        dst_ref.at[pl.ds(base + nfull * step, rem)])


def _make_agg(N, D, n_real):
    npw = N // _NS
    cpw = _NG * _GB
    mesh = plsc.VectorSubcoreMesh(core_axis_name="c", subcore_axis_name="s")
    out_type = [jax.ShapeDtypeStruct((N, D), jnp.float32),
                jax.ShapeDtypeStruct((N, D), jnp.float32)]
    scratch = (
        [pltpu.VMEM((_GB, _B), jnp.int32)] * 2
        + [pltpu.VMEM((_B, D), jnp.float32)] * 2
        + [pltpu.VMEM_SHARED((N, D), jnp.float32)]
        + [pltpu.SemaphoreType.DMA] * 2
    )

    def body(src_hbm, dst_hbm, h_hbm, out0, out1,
             src_v, dst_v, buf0, buf1, acc, sem0, sem1):
        cid = lax.axis_index("c")
        sid = lax.axis_index("s")
        wid = cid * _NS + sid

        _zero_vmem(buf0, _B, D)
        _copy_tiles(buf0, acc, sid * npw, npw, _B)
        plsc.subcore_barrier()

        def gather(j, buf, sem):
            pltpu.async_copy(h_hbm.at[src_v.at[j]], buf, sem)

        def wait_g(buf, sem):
            pltpu.make_async_copy(h_hbm.at[src_v.at[0]], buf, sem).wait()

        def scat(j, buf):
            pltpu.sync_copy(buf, acc.at[dst_v.at[j]], add=True)

        @pl.loop(0, _NG)
        def _(g):
            base = wid * cpw + g * _GB

            @pl.when(base < n_real)
            def _():
                pltpu.sync_copy(src_hbm.at[pl.ds(base, _GB)], src_v)
                pltpu.sync_copy(dst_hbm.at[pl.ds(base, _GB)], dst_v)
                gather(0, buf0, sem0)
                gather(1, buf1, sem1)

                @pl.loop(0, (_GB - 2) // 2)
                def _(t):
                    j = 2 * t
                    wait_g(buf0, sem0)
                    scat(j, buf0)
                    gather(j + 2, buf0, sem0)
                    wait_g(buf1, sem1)
                    scat(j + 1, buf1)
                    gather(j + 3, buf1, sem1)

                wait_g(buf0, sem0)
                scat(_GB - 2, buf0)
                wait_g(buf1, sem1)
                scat(_GB - 1, buf1)

        plsc.subcore_barrier()

        @pl.when(cid == 0)
        def _():
            pltpu.sync_copy(acc.at[pl.ds(sid * npw, npw)],
                            out0.at[pl.ds(sid * npw, npw)])

        @pl.when(cid == 1)
        def _():
            pltpu.sync_copy(acc.at[pl.ds(sid * npw, npw)],
                            out1.at[pl.ds(sid * npw, npw)])

    return pl.kernel(body, out_type=out_type, mesh=mesh,
                     scratch_types=scratch, compiler_params=_SC_PARAMS)


def _make_cnt(N, n_real):
    npw = N // _NS
    cpw = _NG * _GB
    mesh = plsc.VectorSubcoreMesh(core_axis_name="c", subcore_axis_name="s")
    out_type = [jax.ShapeDtypeStruct((N, 16), jnp.float32),
                jax.ShapeDtypeStruct((N, 16), jnp.float32)]
    scratch = (
        [pltpu.VMEM((_GB, _B), jnp.int32),
         pltpu.VMEM((_B, 16), jnp.float32),
         pltpu.VMEM((_B, 16), jnp.float32),
         pltpu.VMEM_SHARED((N, 16), jnp.float32)]
        + [pltpu.SemaphoreType.DMA] * 2
    )

    def body(dst_hbm, out0, out1, dst_v, ones_v, zero_v, cnt, sem0, sem1):
        cid = lax.axis_index("c")
        sid = lax.axis_index("s")
        wid = cid * _NS + sid

        _zero_vmem(zero_v, _B, 16)
        one = jnp.ones((16,), jnp.float32)

        @pl.loop(0, _B)
        def _(r):
            ones_v[r, pl.ds(0, 16)] = one

        _copy_tiles(zero_v, cnt, sid * npw, npw, _B)
        plsc.subcore_barrier()

        def scat(j, sem):
            pltpu.async_copy(ones_v, cnt.at[dst_v.at[j]], sem, add=True)

        def wait_s(sem):
            pltpu.make_async_copy(ones_v, cnt.at[dst_v.at[0]], sem).wait()

        @pl.loop(0, _NG)
        def _(g):
            base = wid * cpw + g * _GB

            @pl.when(base < n_real)
            def _():
                pltpu.sync_copy(dst_hbm.at[pl.ds(base, _GB)], dst_v)
                scat(0, sem0)
                scat(1, sem1)

                @pl.loop(0, (_GB - 2) // 2)
                def _(t):
                    j = 2 * t
                    wait_s(sem0)
                    scat(j + 2, sem0)
                    wait_s(sem1)
                    scat(j + 3, sem1)

                wait_s(sem0)
                wait_s(sem1)

        plsc.subcore_barrier()

        @pl.when(cid == 0)
        def _():
            pltpu.sync_copy(cnt.at[pl.ds(sid * npw, npw)],
                            out0.at[pl.ds(sid * npw, npw)])

        @pl.when(cid == 1)
        def _():
            pltpu.sync_copy(cnt.at[pl.ds(sid * npw, npw)],
                            out1.at[pl.ds(sid * npw, npw)])

    return pl.kernel(body, out_type=out_type, mesh=mesh,
                     scratch_types=scratch, compiler_params=_SC_PARAMS)


def _mm_body(a0_ref, a1_ref, c0_ref, c1_ref, h_ref, wlr_ref, bl_ref,
             pre_ref, act_ref):
    s = a0_ref[...] + a1_ref[...]
    cnt = c0_ref[...] + c1_ref[...]
    r = 1.0 / jnp.maximum(cnt[:, 0:1], 1.0)
    m = s * r
    mh = jnp.concatenate([m, h_ref[...]], axis=1)
    pre = (jnp.dot(mh, wlr_ref[...], preferred_element_type=jnp.float32)
           + bl_ref[...])
    pre_ref[...] = pre
    act_ref[...] = jnp.maximum(pre, 0.0)


def _mm(a0, a1, c0, c1, h, Wl, Wr, bl):
    N, D = h.shape
    R = 1000
    wlr = jnp.concatenate([Wl, Wr], axis=0)
    return pl.pallas_call(
        _mm_body,
        grid=(N // R,),
        in_specs=[
            pl.BlockSpec((R, D), lambda i: (i, 0)),
            pl.BlockSpec((R, D), lambda i: (i, 0)),
            pl.BlockSpec((R, 16), lambda i: (i, 0)),
            pl.BlockSpec((R, 16), lambda i: (i, 0)),
            pl.BlockSpec((R, D), lambda i: (i, 0)),
            pl.BlockSpec((2 * D, D), lambda i: (0, 0)),
            pl.BlockSpec((1, D), lambda i: (0, 0)),
        ],
        out_specs=[pl.BlockSpec((R, D), lambda i: (i, 0)),
                   pl.BlockSpec((R, D), lambda i: (i, 0))],
        out_shape=[jax.ShapeDtypeStruct((N, D), jnp.float32),
                   jax.ShapeDtypeStruct((N, D), jnp.float32)],
    )(a0, a1, c0, c1, h, wlr, bl.reshape(1, D))


def kernel(x, edge_index, Wl0, bl0, Wr0, Wl1, bl1, Wr1, Wl2, bl2, Wr2):
    N, D = x.shape
    E = edge_index.shape[1]
    n_real = E // _B
    src = edge_index[0].reshape(n_real, _B)
    dst = edge_index[1].reshape(n_real, _B)

    agg = _make_agg(N, D, n_real)
    cnt = _make_cnt(N, n_real)

    c0, c1 = cnt(dst)
    a0, a1 = agg(src, dst, x)
    pre0, h1 = _mm(a0, a1, c0, c1, x, Wl0, Wr0, bl0)
    a0, a1 = agg(src, dst, h1)
    pre1, h2 = _mm(a0, a1, c0, c1, h1, Wl1, Wr1, bl1)
    a0, a1 = agg(src, dst, h2)
    pre2, _ = _mm(a0, a1, c0, c1, h2, Wl2, Wr2, bl2)
    return (pre2, pre1)


# speedup vs baseline: 3.5928x; 1.0137x over previous
import jax
import jax.numpy as jnp
from jax import lax
from jax.experimental import pallas as pl
from jax.experimental.pallas import tpu as pltpu
from jax.experimental.pallas import tpu_sc as plsc

_NC = 2
_NS = 16
_NW = _NC * _NS
_B = 128
_GB = 20
_NG = 4

_SC_PARAMS = pltpu.CompilerParams(use_tc_tiling_on_sc=False)


def _fill_vmem(buf, rows, width, vec):
    @pl.loop(0, rows)
    def _(r):
        for k in range(width // 16):
            buf[r, pl.ds(16 * k, 16)] = vec


def _copy_tiles(src_buf, dst_ref, base, total, step):
    nfull = total // step
    rem = total - nfull * step

    @pl.loop(0, nfull)
    def _(i):
        pltpu.sync_copy(src_buf, dst_ref.at[pl.ds(base + i * step, step)])

    if rem:
        pltpu.sync_copy(src_buf.at[pl.ds(0, rem)],
                        dst_ref.at[pl.ds(base + nfull * step, rem)])


def _make_agg(N, D, n_real, with_cnt):
    npw = N // _NS
    cpw = _NG * _GB
    mesh = plsc.VectorSubcoreMesh(core_axis_name="c", subcore_axis_name="s")
    out_type = [jax.ShapeDtypeStruct((N, D), jnp.float32),
                jax.ShapeDtypeStruct((N, D), jnp.float32)]
    scratch = (
        [pltpu.VMEM((_GB, _B), jnp.int32)] * 2
        + [pltpu.VMEM((_B, D), jnp.float32)] * 2
        + [pltpu.VMEM_SHARED((N, D), jnp.float32)]
        + [pltpu.SemaphoreType.DMA] * 2
    )
    if with_cnt:
        out_type += [jax.ShapeDtypeStruct((N, 16), jnp.float32),
                     jax.ShapeDtypeStruct((N, 16), jnp.float32)]
        scratch += [pltpu.VMEM((_B, 16), jnp.float32),
                    pltpu.VMEM((16, 16), jnp.float32),
                    pltpu.VMEM_SHARED((N, 16), jnp.float32),
                    pltpu.SemaphoreType.DMA,
                    pltpu.SemaphoreType.DMA]

    def body(src_hbm, dst_hbm, h_hbm, out0, out1, *rest):
        if with_cnt:
            (cnt_o0, cnt_o1, src_v, dst_v, buf0, buf1, acc, sem0, sem1,
             ones_v, z16_v, cnt, csem0, csem1) = rest
        else:
            (src_v, dst_v, buf0, buf1, acc, sem0, sem1) = rest
        cid = lax.axis_index("c")
        sid = lax.axis_index("s")
        wid = cid * _NS + sid

        _fill_vmem(buf0, _B, D, jnp.zeros((16,), jnp.float32))
        _copy_tiles(buf0, acc, sid * npw, npw, _B)
        if with_cnt:
            _fill_vmem(ones_v, _B, 16, jnp.ones((16,), jnp.float32))
            _fill_vmem(z16_v, 16, 16, jnp.zeros((16,), jnp.float32))
            _copy_tiles(z16_v, cnt, sid * npw, npw, 16)
        plsc.subcore_barrier()

        def gather(j, buf, sem):
            pltpu.async_copy(h_hbm.at[src_v.at[j]], buf, sem)
            if with_cnt:
                csem = csem0 if buf is buf0 else csem1
                pltpu.async_copy(ones_v, cnt.at[dst_v.at[j]], csem, add=True)

        def wait_g(buf, sem):
            pltpu.make_async_copy(h_hbm.at[src_v.at[0]], buf, sem).wait()
            if with_cnt:
                csem = csem0 if buf is buf0 else csem1
                pltpu.make_async_copy(ones_v, cnt.at[dst_v.at[0]],
                                      csem).wait()

        def scat(j, buf):
            pltpu.sync_copy(buf, acc.at[dst_v.at[j]], add=True)

        @pl.loop(0, _NG)
        def _(g):
            base = wid * cpw + g * _GB

            @pl.when(base < n_real)
            def _():
                pltpu.sync_copy(src_hbm.at[pl.ds(base, _GB)], src_v)
                pltpu.sync_copy(dst_hbm.at[pl.ds(base, _GB)], dst_v)
                gather(0, buf0, sem0)
                gather(1, buf1, sem1)

                @pl.loop(0, (_GB - 2) // 2)
                def _(t):
                    j = 2 * t
                    wait_g(buf0, sem0)
                    scat(j, buf0)
                    gather(j + 2, buf0, sem0)
                    wait_g(buf1, sem1)
                    scat(j + 1, buf1)
                    gather(j + 3, buf1, sem1)

                wait_g(buf0, sem0)
                scat(_GB - 2, buf0)
                wait_g(buf1, sem1)
                scat(_GB - 1, buf1)

        plsc.subcore_barrier()

        @pl.when(cid == 0)
        def _():
            pltpu.sync_copy(acc.at[pl.ds(sid * npw, npw)],
                            out0.at[pl.ds(sid * npw, npw)])
            if with_cnt:
                pltpu.sync_copy(cnt.at[pl.ds(sid * npw, npw)],
                                cnt_o0.at[pl.ds(sid * npw, npw)])

        @pl.when(cid == 1)
        def _():
            pltpu.sync_copy(acc.at[pl.ds(sid * npw, npw)],
                            out1.at[pl.ds(sid * npw, npw)])
            if with_cnt:
                pltpu.sync_copy(cnt.at[pl.ds(sid * npw, npw)],
                                cnt_o1.at[pl.ds(sid * npw, npw)])

    return pl.kernel(body, out_type=out_type, mesh=mesh,
                     scratch_types=scratch, compiler_params=_SC_PARAMS)


def _mm_body(a0_ref, a1_ref, c0_ref, c1_ref, h_ref, wlr_ref, bl_ref,
             pre_ref, act_ref):
    s = a0_ref[...] + a1_ref[...]
    cnt = c0_ref[...] + c1_ref[...]
    r = 1.0 / jnp.maximum(cnt[:, 0:1], 1.0)
    m = s * r
    mh = jnp.concatenate([m, h_ref[...]], axis=1)
    pre = (jnp.dot(mh, wlr_ref[...], preferred_element_type=jnp.float32)
           + bl_ref[...])
    pre_ref[...] = pre
    act_ref[...] = jnp.maximum(pre, 0.0)


def _mm(a0, a1, c0, c1, h, Wl, Wr, bl):
    N, D = h.shape
    R = 1000
    wlr = jnp.concatenate([Wl, Wr], axis=0)
    return pl.pallas_call(
        _mm_body,
        grid=(N // R,),
        in_specs=[
            pl.BlockSpec((R, D), lambda i: (i, 0)),
            pl.BlockSpec((R, D), lambda i: (i, 0)),
            pl.BlockSpec((R, 16), lambda i: (i, 0)),
            pl.BlockSpec((R, 16), lambda i: (i, 0)),
            pl.BlockSpec((R, D), lambda i: (i, 0)),
            pl.BlockSpec((2 * D, D), lambda i: (0, 0)),
            pl.BlockSpec((1, D), lambda i: (0, 0)),
        ],
        out_specs=[pl.BlockSpec((R, D), lambda i: (i, 0)),
                   pl.BlockSpec((R, D), lambda i: (i, 0))],
        out_shape=[jax.ShapeDtypeStruct((N, D), jnp.float32),
                   jax.ShapeDtypeStruct((N, D), jnp.float32)],
    )(a0, a1, c0, c1, h, wlr, bl.reshape(1, D))


def kernel(x, edge_index, Wl0, bl0, Wr0, Wl1, bl1, Wr1, Wl2, bl2, Wr2):
    N, D = x.shape
    E = edge_index.shape[1]
    n_real = E // _B
    src = edge_index[0].reshape(n_real, _B)
    dst = edge_index[1].reshape(n_real, _B)

    agg_cnt = _make_agg(N, D, n_real, True)
    agg = _make_agg(N, D, n_real, False)

    a0, a1, c0, c1 = agg_cnt(src, dst, x)
    pre0, h1 = _mm(a0, a1, c0, c1, x, Wl0, Wr0, bl0)
    a0, a1 = agg(src, dst, h1)
    pre1, h2 = _mm(a0, a1, c0, c1, h1, Wl1, Wr1, bl1)
    a0, a1 = agg(src, dst, h2)
    pre2, _ = _mm(a0, a1, c0, c1, h2, Wl2, Wr2, bl2)
    return (pre2, pre1)
